# baseline (speedup 1.0000x reference)
import functools

import jax
import jax.numpy as jnp
from jax import lax
from jax.experimental import pallas as pl
from jax.experimental.pallas import tpu as pltpu
from jax.experimental.pallas import tpu_sc as plsc

N_NODES = 10000
D = 128
E = 320000

NC = 2
NS = 16
NW = NC * NS
EPT = E // NW
K = 128
FULL = EPT // K
TAIL = EPT - FULL * K
ZB = 1000
NZ = N_NODES // ZB

_MESH = plsc.VectorSubcoreMesh(core_axis_name="c", subcore_axis_name="s")


NCH = 80
EPP = NCH * K
PAD = EPP - EPT
NPAD = N_NODES + K
QD = 8


def _deg_body(col3_hbm, ones_hbm, zeros_hbm, degp_hbm, cidx_v, ones_v,
              stage_v, deg_sh, sem):
    cid = lax.axis_index("c")
    sid = lax.axis_index("s")
    wid = cid * NS + sid

    @pl.when(sid < NZ)
    def _():
        pltpu.sync_copy(zeros_hbm, stage_v)
        pltpu.sync_copy(stage_v, deg_sh.at[pl.ds(sid * ZB, ZB)])

    @pl.when(sid == NZ)
    def _():
        pltpu.sync_copy(zeros_hbm.at[pl.ds(0, K)], stage_v.at[pl.ds(0, K)])
        pltpu.sync_copy(stage_v.at[pl.ds(0, K)],
                        deg_sh.at[pl.ds(N_NODES, K)])

    pltpu.sync_copy(ones_hbm, ones_v)
    pltpu.sync_copy(col3_hbm.at[wid], cidx_v)
    plsc.subcore_barrier()

    def body(c, carry):
        pltpu.async_copy(ones_v, deg_sh.at[cidx_v.at[c]], sem, add=True)

        @pl.when(c >= QD)
        def _():
            pltpu.make_async_copy(ones_v, deg_sh.at[cidx_v.at[c - QD]],
                                  sem).wait()

        return carry

    lax.fori_loop(0, NCH, body, 0)

    def drain(c, carry):
        pltpu.make_async_copy(ones_v, deg_sh.at[cidx_v.at[c]], sem).wait()
        return carry

    lax.fori_loop(NCH - QD, NCH, drain, 0)

    plsc.subcore_barrier()

    @pl.when(sid < NZ)
    def _():
        pltpu.sync_copy(deg_sh.at[pl.ds(sid * ZB, ZB)], stage_v)
        pltpu.sync_copy(stage_v,
                        degp_hbm.at[pl.ds(cid * N_NODES + sid * ZB, ZB)])


_deg_call = pl.kernel(
    _deg_body,
    out_type=jax.ShapeDtypeStruct((NC * N_NODES,), jnp.float32),
    mesh=_MESH,
    scratch_types=[
        pltpu.VMEM((NCH, K), jnp.int32),
        pltpu.VMEM((K,), jnp.float32),
        pltpu.VMEM((ZB,), jnp.float32),
        pltpu.VMEM_SHARED((NPAD,), jnp.float32),
        pltpu.SemaphoreType.DMA,
    ],
)


WB = 40


def _acc_body(g_hbm, row_hbm, col_hbm, zrows_hbm, accp_hbm, ridx_a, cidx_a,
              ridx_b, cidx_b, ridx_t, cidx_t, rows_a, rows_b, rows_t, zb_v,
              acc_sh, g_a, g_b, s_a, s_b, r_a, r_b, c_a, c_b):
    cid = lax.axis_index("c")
    sid = lax.axis_index("s")
    base = (cid * NS + sid) * EPT

    @pl.when(sid < NZ)
    def _():
        pltpu.sync_copy(zrows_hbm, zb_v)

        def zbody(j, carry):
            pltpu.sync_copy(zb_v, acc_sh.at[pl.ds(sid * ZB + j * WB, WB)])
            return carry

        lax.fori_loop(0, ZB // WB, zbody, 0)

    plsc.subcore_barrier()

    def fire_r(c, ridx, sem):
        pltpu.async_copy(row_hbm.at[pl.ds(base + c * K, K)], ridx, sem)

    def wait_r(c, ridx, sem):
        pltpu.make_async_copy(row_hbm.at[pl.ds(base + c * K, K)], ridx,
                              sem).wait()

    def fire_c(c, cidx, sem):
        pltpu.async_copy(col_hbm.at[pl.ds(base + c * K, K)], cidx, sem)

    def wait_c(c, cidx, sem):
        pltpu.make_async_copy(col_hbm.at[pl.ds(base + c * K, K)], cidx,
                              sem).wait()

    def fire_g(rows, ridx, sem):
        pltpu.async_copy(g_hbm.at[ridx], rows, sem)

    def wait_g(rows, ridx, sem):
        pltpu.make_async_copy(g_hbm.at[ridx], rows, sem).wait()

    def fire_s(rows, cidx, sem):
        pltpu.async_copy(rows, acc_sh.at[cidx], sem, add=True)

    def wait_s(rows, cidx, sem):
        pltpu.make_async_copy(rows, acc_sh.at[cidx], sem).wait()

    pltpu.sync_copy(row_hbm.at[pl.ds(base, K)], ridx_a)
    pltpu.sync_copy(col_hbm.at[pl.ds(base, K)], cidx_a)
    fire_g(rows_a, ridx_a, g_a)
    fire_r(1, ridx_b, r_b)
    wait_g(rows_a, ridx_a, g_a)
    fire_s(rows_a, cidx_a, s_a)
    fire_c(1, cidx_b, c_b)
    wait_r(1, ridx_b, r_b)
    fire_g(rows_b, ridx_b, g_b)
    fire_r(2, ridx_a, r_a)
    wait_c(1, cidx_b, c_b)

    def step(a, rows_x, ridx_x, cidx_x, g_x, s_x, r_x, c_x,
             rows_y, ridx_y, cidx_y, g_y, s_y, r_y, c_y, last):
        wait_g(rows_x, ridx_x, g_x)
        fire_s(rows_x, cidx_x, s_x)
        wait_s(rows_y, cidx_y, s_y)

        @pl.when(a + 1 < FULL)
        def _():
            fire_c(a + 1, cidx_y, c_y)
            wait_r(a + 1, ridx_y, r_y)
            fire_g(rows_y, ridx_y, g_y)

            @pl.when(a + 2 < FULL)
            def _():
                fire_r(a + 2, ridx_x, r_x)

            wait_c(a + 1, cidx_y, c_y)

    A = (rows_a, ridx_a, cidx_a, g_a, s_a, r_a, c_a)
    B = (rows_b, ridx_b, cidx_b, g_b, s_b, r_b, c_b)

    def body(i, carry):
        a = 2 * i + 1
        step(a, *B, *A, False)
        step(a + 1, *A, *B, False)
        return carry

    lax.fori_loop(0, (FULL - 2) // 2, body, 0)
    step(FULL - 1, *B, *A, True)
    wait_s(rows_b, cidx_b, s_b)
    e0 = base + FULL * K
    pltpu.sync_copy(row_hbm.at[pl.ds(e0, TAIL)], ridx_t)
    pltpu.sync_copy(col_hbm.at[pl.ds(e0, TAIL)], cidx_t)
    pltpu.async_copy(g_hbm.at[ridx_t], rows_t, g_a)
    pltpu.make_async_copy(g_hbm.at[ridx_t], rows_t, g_a).wait()
    pltpu.sync_copy(rows_t, acc_sh.at[cidx_t], add=True)

    plsc.subcore_barrier()

    @pl.when(sid < NZ)
    def _():
        def wbody(j, carry):
            r0 = sid * ZB + j * WB
            pltpu.sync_copy(acc_sh.at[pl.ds(r0, WB)], zb_v)
            pltpu.sync_copy(zb_v, accp_hbm.at[cid, pl.ds(r0, WB)])
            return carry

        lax.fori_loop(0, ZB // WB, wbody, 0)


_acc_call = pl.kernel(
    _acc_body,
    out_type=jax.ShapeDtypeStruct((NC, N_NODES, D), jnp.float32),
    mesh=_MESH,
    scratch_types=[
        pltpu.VMEM((K,), jnp.int32),
        pltpu.VMEM((K,), jnp.int32),
        pltpu.VMEM((K,), jnp.int32),
        pltpu.VMEM((K,), jnp.int32),
        pltpu.VMEM((TAIL,), jnp.int32),
        pltpu.VMEM((TAIL,), jnp.int32),
        pltpu.VMEM((K, D), jnp.float32),
        pltpu.VMEM((K, D), jnp.float32),
        pltpu.VMEM((TAIL, D), jnp.float32),
        pltpu.VMEM((WB, D), jnp.float32),
        pltpu.VMEM_SHARED((N_NODES, D), jnp.float32),
        pltpu.SemaphoreType.DMA,
        pltpu.SemaphoreType.DMA,
        pltpu.SemaphoreType.DMA,
        pltpu.SemaphoreType.DMA,
        pltpu.SemaphoreType.DMA,
        pltpu.SemaphoreType.DMA,
        pltpu.SemaphoreType.DMA,
        pltpu.SemaphoreType.DMA,
    ],
)


BLK = 1000


def _lin_body(x_ref, w_ref, bl_ref, degp_ref, g_ref):
    deg = degp_ref[:, 0] + degp_ref[:, 1]
    dis = jnp.where(deg > 0.0, lax.rsqrt(deg), 0.0)
    h = jnp.dot(x_ref[...], w_ref[...].T,
                preferred_element_type=jnp.float32) + bl_ref[...]
    g_ref[...] = h * dis[:, None]


_lin_call = pl.pallas_call(
    _lin_body,
    grid=(N_NODES // BLK,),
    in_specs=[
        pl.BlockSpec((BLK, D), lambda i: (i, 0)),
        pl.BlockSpec((D, D), lambda i: (0, 0)),
        pl.BlockSpec((1, D), lambda i: (0, 0)),
        pl.BlockSpec((BLK, NC), lambda i: (i, 0)),
    ],
    out_specs=pl.BlockSpec((BLK, D), lambda i: (i, 0)),
    out_shape=jax.ShapeDtypeStruct((N_NODES, D), jnp.float32),
)


def _out_body(accp_ref, degp_ref, bias_ref, out_ref):
    acc = accp_ref[0] + accp_ref[1]
    deg = degp_ref[:, 0] + degp_ref[:, 1]
    dis = jnp.where(deg > 0.0, lax.rsqrt(deg), 0.0)
    out_ref[...] = jnp.maximum(acc * dis[:, None] + bias_ref[...], 0.0)


_out_call = pl.pallas_call(
    _out_body,
    grid=(N_NODES // BLK,),
    in_specs=[
        pl.BlockSpec((NC, BLK, D), lambda i: (0, i, 0)),
        pl.BlockSpec((BLK, NC), lambda i: (i, 0)),
        pl.BlockSpec((1, D), lambda i: (0, 0)),
    ],
    out_specs=pl.BlockSpec((BLK, D), lambda i: (i, 0)),
    out_shape=jax.ShapeDtypeStruct((N_NODES, D), jnp.float32),
)


@jax.jit
def kernel(x, edge_index, W, b_lin, bias):
    row = edge_index[0]
    col = edge_index[1]
    ones_k = jnp.ones((K,), jnp.float32)
    zeros_n = jnp.zeros((ZB,), jnp.float32)
    zrows = jnp.zeros((WB, D), jnp.float32)

    padcol = N_NODES + (jnp.arange(PAD, dtype=jnp.int32) % K)
    colp = jnp.concatenate(
        [col.reshape(NW, EPT), jnp.broadcast_to(padcol, (NW, PAD))], axis=1)
    col3 = colp.reshape(NW, NCH, K)

    degp = _deg_call(col3, ones_k, zeros_n)
    degp_t = degp.reshape(NC, N_NODES).T
    g = _lin_call(x, W, b_lin.reshape(1, D), degp_t)
    accp = _acc_call(g, row, col, zrows)
    out = _out_call(accp, degp_t, bias.reshape(1, D))
    return out


# speedup vs baseline: 1.0544x; 1.0544x over previous
import functools

import jax
import jax.numpy as jnp
from jax import lax
from jax.experimental import pallas as pl
from jax.experimental.pallas import tpu as pltpu
from jax.experimental.pallas import tpu_sc as plsc

N_NODES = 10000
D = 128
E = 320000

NC = 2
NS = 16
NW = NC * NS
EPT = E // NW
K = 128
FULL = EPT // K
TAIL = EPT - FULL * K
ZB = 1000
NZ = N_NODES // ZB

_MESH = plsc.VectorSubcoreMesh(core_axis_name="c", subcore_axis_name="s")


NCH = 80
EPP = NCH * K
PAD = EPP - EPT
NPAD = N_NODES + K
QD = 8


def _deg_body(col3_hbm, ones_hbm, zeros_hbm, degp_hbm, cidx_v, ones_v,
              stage_v, deg_sh, sem):
    cid = lax.axis_index("c")
    sid = lax.axis_index("s")
    wid = cid * NS + sid

    @pl.when(sid < NZ)
    def _():
        pltpu.sync_copy(zeros_hbm, stage_v)
        pltpu.sync_copy(stage_v, deg_sh.at[pl.ds(sid * ZB, ZB)])

    @pl.when(sid == NZ)
    def _():
        pltpu.sync_copy(zeros_hbm.at[pl.ds(0, K)], stage_v.at[pl.ds(0, K)])
        pltpu.sync_copy(stage_v.at[pl.ds(0, K)],
                        deg_sh.at[pl.ds(N_NODES, K)])

    pltpu.sync_copy(ones_hbm, ones_v)
    pltpu.sync_copy(col3_hbm.at[wid], cidx_v)
    plsc.subcore_barrier()

    def body(c, carry):
        pltpu.async_copy(ones_v, deg_sh.at[cidx_v.at[c]], sem, add=True)

        @pl.when(c >= QD)
        def _():
            pltpu.make_async_copy(ones_v, deg_sh.at[cidx_v.at[c - QD]],
                                  sem).wait()

        return carry

    lax.fori_loop(0, NCH, body, 0)

    def drain(c, carry):
        pltpu.make_async_copy(ones_v, deg_sh.at[cidx_v.at[c]], sem).wait()
        return carry

    lax.fori_loop(NCH - QD, NCH, drain, 0)

    plsc.subcore_barrier()

    @pl.when(sid < NZ)
    def _():
        pltpu.sync_copy(deg_sh.at[pl.ds(sid * ZB, ZB)], stage_v)
        pltpu.sync_copy(stage_v,
                        degp_hbm.at[pl.ds(cid * N_NODES + sid * ZB, ZB)])


_deg_call = pl.kernel(
    _deg_body,
    out_type=jax.ShapeDtypeStruct((NC * N_NODES,), jnp.float32),
    mesh=_MESH,
    scratch_types=[
        pltpu.VMEM((NCH, K), jnp.int32),
        pltpu.VMEM((K,), jnp.float32),
        pltpu.VMEM((ZB,), jnp.float32),
        pltpu.VMEM_SHARED((NPAD,), jnp.float32),
        pltpu.SemaphoreType.DMA,
    ],
)


WB = 40


def _acc_body(g_hbm, row_hbm, col_hbm, zrows_hbm, accp_hbm, ridx_a, cidx_a,
              ridx_b, cidx_b, ridx_c, cidx_c, ridx_t, cidx_t, rows_a,
              rows_b, rows_c, acc_sh, g_a, g_b, g_c, i_a, i_b, i_c):
    cid = lax.axis_index("c")
    sid = lax.axis_index("s")
    base = (cid * NS + sid) * EPT

    @pl.when(sid < NZ)
    def _():
        pltpu.sync_copy(zrows_hbm, rows_a.at[pl.ds(0, WB)])

        def zbody(j, carry):
            pltpu.sync_copy(rows_a.at[pl.ds(0, WB)],
                            acc_sh.at[pl.ds(sid * ZB + j * WB, WB)])
            return carry

        lax.fori_loop(0, ZB // WB, zbody, 0)

    plsc.subcore_barrier()

    def load_idx(c, ridx, cidx):
        e0 = base + c * K
        pltpu.sync_copy(row_hbm.at[pl.ds(e0, K)], ridx)
        pltpu.sync_copy(col_hbm.at[pl.ds(e0, K)], cidx)

    def fire_idx(c, ridx, cidx, sem):
        e0 = base + c * K
        pltpu.async_copy(row_hbm.at[pl.ds(e0, K)], ridx, sem)
        pltpu.async_copy(col_hbm.at[pl.ds(e0, K)], cidx, sem)

    def wait_idx(c, ridx, cidx, sem):
        e0 = base + c * K
        pltpu.make_async_copy(row_hbm.at[pl.ds(e0, K)], ridx, sem).wait()
        pltpu.make_async_copy(col_hbm.at[pl.ds(e0, K)], cidx, sem).wait()

    load_idx(0, ridx_a, cidx_a)
    load_idx(1, ridx_b, cidx_b)
    pltpu.async_copy(g_hbm.at[ridx_a], rows_a, g_a)
    pltpu.async_copy(g_hbm.at[ridx_b], rows_b, g_b)
    fire_idx(2, ridx_c, cidx_c, i_c)

    def step(a, rows_x, ridx_x, cidx_x, g_x, i_x,
             rows_z, ridx_z, cidx_z, g_z, i_z):
        pltpu.make_async_copy(g_hbm.at[ridx_x], rows_x, g_x).wait()

        @pl.when(a + 2 < FULL)
        def _():
            wait_idx(a + 2, ridx_z, cidx_z, i_z)
            pltpu.async_copy(g_hbm.at[ridx_z], rows_z, g_z)

        pltpu.sync_copy(rows_x, acc_sh.at[cidx_x], add=True)

        @pl.when(a + 3 < FULL)
        def _():
            fire_idx(a + 3, ridx_x, cidx_x, i_x)

    A = (rows_a, ridx_a, cidx_a, g_a, i_a)
    B = (rows_b, ridx_b, cidx_b, g_b, i_b)
    C = (rows_c, ridx_c, cidx_c, g_c, i_c)

    def body(i, carry):
        a = 3 * i
        step(a, *A, *C)
        step(a + 1, *B, *A)
        step(a + 2, *C, *B)
        return carry

    lax.fori_loop(0, FULL // 3, body, 0)
    e0 = base + FULL * K
    pltpu.sync_copy(row_hbm.at[pl.ds(e0, TAIL)], ridx_t)
    pltpu.sync_copy(col_hbm.at[pl.ds(e0, TAIL)], cidx_t)
    pltpu.async_copy(g_hbm.at[ridx_t], rows_a.at[pl.ds(0, TAIL)], g_a)
    pltpu.make_async_copy(g_hbm.at[ridx_t], rows_a.at[pl.ds(0, TAIL)],
                          g_a).wait()
    pltpu.sync_copy(rows_a.at[pl.ds(0, TAIL)], acc_sh.at[cidx_t], add=True)

    plsc.subcore_barrier()

    @pl.when(sid < NZ)
    def _():
        def wbody(j, carry):
            r0 = sid * ZB + j * WB
            pltpu.sync_copy(acc_sh.at[pl.ds(r0, WB)], rows_a.at[pl.ds(0, WB)])
            pltpu.sync_copy(rows_a.at[pl.ds(0, WB)],
                            accp_hbm.at[cid, pl.ds(r0, WB)])
            return carry

        lax.fori_loop(0, ZB // WB, wbody, 0)


_acc_call = pl.kernel(
    _acc_body,
    out_type=jax.ShapeDtypeStruct((NC, N_NODES, D), jnp.float32),
    mesh=_MESH,
    scratch_types=[
        pltpu.VMEM((K,), jnp.int32),
        pltpu.VMEM((K,), jnp.int32),
        pltpu.VMEM((K,), jnp.int32),
        pltpu.VMEM((K,), jnp.int32),
        pltpu.VMEM((K,), jnp.int32),
        pltpu.VMEM((K,), jnp.int32),
        pltpu.VMEM((TAIL,), jnp.int32),
        pltpu.VMEM((TAIL,), jnp.int32),
        pltpu.VMEM((K, D), jnp.float32),
        pltpu.VMEM((K, D), jnp.float32),
        pltpu.VMEM((K, D), jnp.float32),
        pltpu.VMEM_SHARED((N_NODES, D), jnp.float32),
        pltpu.SemaphoreType.DMA,
        pltpu.SemaphoreType.DMA,
        pltpu.SemaphoreType.DMA,
        pltpu.SemaphoreType.DMA,
        pltpu.SemaphoreType.DMA,
        pltpu.SemaphoreType.DMA,
    ],
)


BLK = 1000


def _lin_body(x_ref, w_ref, bl_ref, degp_ref, g_ref):
    deg = degp_ref[:, 0] + degp_ref[:, 1]
    dis = jnp.where(deg > 0.0, lax.rsqrt(deg), 0.0)
    h = jnp.dot(x_ref[...], w_ref[...].T,
                preferred_element_type=jnp.float32) + bl_ref[...]
    g_ref[...] = h * dis[:, None]


_lin_call = pl.pallas_call(
    _lin_body,
    grid=(N_NODES // BLK,),
    in_specs=[
        pl.BlockSpec((BLK, D), lambda i: (i, 0)),
        pl.BlockSpec((D, D), lambda i: (0, 0)),
        pl.BlockSpec((1, D), lambda i: (0, 0)),
        pl.BlockSpec((BLK, NC), lambda i: (i, 0)),
    ],
    out_specs=pl.BlockSpec((BLK, D), lambda i: (i, 0)),
    out_shape=jax.ShapeDtypeStruct((N_NODES, D), jnp.float32),
)


def _out_body(accp_ref, degp_ref, bias_ref, out_ref):
    acc = accp_ref[0] + accp_ref[1]
    deg = degp_ref[:, 0] + degp_ref[:, 1]
    dis = jnp.where(deg > 0.0, lax.rsqrt(deg), 0.0)
    out_ref[...] = jnp.maximum(acc * dis[:, None] + bias_ref[...], 0.0)


_out_call = pl.pallas_call(
    _out_body,
    grid=(N_NODES // BLK,),
    in_specs=[
        pl.BlockSpec((NC, BLK, D), lambda i: (0, i, 0)),
        pl.BlockSpec((BLK, NC), lambda i: (i, 0)),
        pl.BlockSpec((1, D), lambda i: (0, 0)),
    ],
    out_specs=pl.BlockSpec((BLK, D), lambda i: (i, 0)),
    out_shape=jax.ShapeDtypeStruct((N_NODES, D), jnp.float32),
)


@jax.jit
def kernel(x, edge_index, W, b_lin, bias):
    row = edge_index[0]
    col = edge_index[1]
    ones_k = jnp.ones((K,), jnp.float32)
    zeros_n = jnp.zeros((ZB,), jnp.float32)
    zrows = jnp.zeros((WB, D), jnp.float32)

    padcol = N_NODES + (jnp.arange(PAD, dtype=jnp.int32) % K)
    colp = jnp.concatenate(
        [col.reshape(NW, EPT), jnp.broadcast_to(padcol, (NW, PAD))], axis=1)
    col3 = colp.reshape(NW, NCH, K)

    degp = _deg_call(col3, ones_k, zeros_n)
    degp_t = degp.reshape(NC, N_NODES).T
    g = _lin_call(x, W, b_lin.reshape(1, D), degp_t)
    accp = _acc_call(g, row, col, zrows)
    out = _out_call(accp, degp_t, bias.reshape(1, D))
    return out
